# baseline (speedup 1.0000x reference)
import functools

import jax
import jax.numpy as jnp
from jax import lax
from jax.experimental import pallas as pl
from jax.experimental.pallas import tpu as pltpu
from jax.experimental.pallas import tpu_sc as plsc

N = 32768
L = 16
NC = 2
NS = 16
SCAN = N // NS
OUT = N // (NC * NS)
I32_MAX = 2**31 - 1

_mesh = plsc.VectorSubcoreMesh(core_axis_name="c", subcore_axis_name="s")


@functools.partial(
    pl.kernel,
    out_type=jax.ShapeDtypeStruct((N,), jnp.float32),
    mesh=_mesh,
    compiler_params=pltpu.CompilerParams(needs_layout_passes=False),
    scratch_types=[
        pltpu.VMEM((SCAN,), jnp.float32),
        pltpu.VMEM((OUT,), jnp.float32),
        pltpu.VMEM((L,), jnp.float32),
        pltpu.VMEM((L,), jnp.int32),
        pltpu.VMEM((NS * L,), jnp.float32),
        pltpu.VMEM((NS * L,), jnp.int32),
        pltpu.VMEM_SHARED((NS * L,), jnp.float32),
        pltpu.VMEM_SHARED((NS * L,), jnp.int32),
    ],
)
def _stargmin_sc(x_hbm, out_hbm, chunk_v, out_v, stage_f, stage_i,
                 allf_v, alli_v, shared_f, shared_i):
    cid = lax.axis_index("c")
    sid = lax.axis_index("s")
    lanes = lax.iota(jnp.int32, L)

    base_s = sid * SCAN
    pltpu.sync_copy(x_hbm.at[pl.ds(base_s, SCAN)], chunk_v)

    def scan_body(j, carry):
        vmin, vidx = carry
        v = chunk_v[pl.ds(j * L, L)]
        gidx = base_s + j * L + lanes
        take = v < vmin
        return jnp.where(take, v, vmin), jnp.where(take, gidx, vidx)

    vmin, vidx = lax.fori_loop(
        0, SCAN // L, scan_body,
        (jnp.full((L,), jnp.inf, jnp.float32), jnp.zeros((L,), jnp.int32)),
    )

    mval = jnp.min(vmin)
    midx = jnp.min(jnp.where(vmin == mval, vidx, I32_MAX))

    stage_f[...] = jnp.full((L,), mval, jnp.float32)
    stage_i[...] = jnp.full((L,), midx, jnp.int32)
    pltpu.sync_copy(stage_f, shared_f.at[pl.ds(sid * L, L)])
    pltpu.sync_copy(stage_i, shared_i.at[pl.ds(sid * L, L)])
    plsc.subcore_barrier()
    pltpu.sync_copy(shared_f, allf_v)
    pltpu.sync_copy(shared_i, alli_v)

    def merge_body(j, carry):
        bvals, bidxs = carry
        take = lanes == j
        return (jnp.where(take, allf_v[pl.ds(j * L, L)], bvals),
                jnp.where(take, alli_v[pl.ds(j * L, L)], bidxs))

    vals16, idxs16 = lax.fori_loop(
        0, NS, merge_body,
        (jnp.full((L,), jnp.inf, jnp.float32), jnp.zeros((L,), jnp.int32)),
    )
    gmin = jnp.min(vals16)
    gidx = jnp.min(jnp.where(vals16 == gmin, idxs16, I32_MAX))

    wid = sid * NC + cid
    base_o = wid * OUT

    def fill_body(j, _):
        pos = base_o + j * L + lanes
        out_v[pl.ds(j * L, L)] = jnp.where(pos == gidx, 1.0, 0.0).astype(
            jnp.float32)
        return 0

    lax.fori_loop(0, OUT // L, fill_body, 0)
    pltpu.sync_copy(out_v, out_hbm.at[pl.ds(base_o, OUT)])


def kernel(x):
    return _stargmin_sc(x.reshape(N)).reshape(1, N)


# speedup vs baseline: 1.1082x; 1.1082x over previous
import functools

import jax
import jax.numpy as jnp
from jax import lax
from jax.experimental import pallas as pl
from jax.experimental.pallas import tpu as pltpu
from jax.experimental.pallas import tpu_sc as plsc

N = 32768
L = 16
NS = 16
SCAN = N // NS
I32_MAX = 2**31 - 1

_mesh = plsc.VectorSubcoreMesh(core_axis_name="c", subcore_axis_name="s",
                               num_cores=1)


@functools.partial(
    pl.kernel,
    out_type=jax.ShapeDtypeStruct((N,), jnp.float32),
    mesh=_mesh,
    compiler_params=pltpu.CompilerParams(needs_layout_passes=False),
    scratch_types=[
        pltpu.VMEM((SCAN,), jnp.float32),
        pltpu.VMEM((2 * L,), jnp.float32),
        pltpu.VMEM((NS * 2 * L,), jnp.float32),
        pltpu.VMEM_SHARED((NS * 2 * L,), jnp.float32),
    ],
)
def _stargmin_sc(x_hbm, out_hbm, chunk_v, stage_v, all_v, shared_v):
    sid = lax.axis_index("s")
    lanes = lax.iota(jnp.int32, L)

    base = sid * SCAN
    pltpu.sync_copy(x_hbm.at[pl.ds(base, SCAN)], chunk_v)

    def scan_body(j, carry):
        vmin, vidx = carry
        v = chunk_v[pl.ds(j * L, L)]
        gidx = base + j * L + lanes
        take = v < vmin
        return jnp.where(take, v, vmin), jnp.where(take, gidx, vidx)

    vmin, vidx = lax.fori_loop(
        0, SCAN // L, scan_body,
        (jnp.full((L,), jnp.inf, jnp.float32), jnp.zeros((L,), jnp.int32)),
        unroll=8,
    )

    mval = jnp.min(vmin)
    midx = jnp.min(jnp.where(vmin == mval, vidx, I32_MAX))

    stage_v[pl.ds(0, L)] = jnp.full((L,), mval, jnp.float32)
    stage_v[pl.ds(L, L)] = jnp.full((L,), midx, jnp.int32).astype(jnp.float32)
    pltpu.sync_copy(stage_v, shared_v.at[pl.ds(sid * 2 * L, 2 * L)])
    plsc.subcore_barrier()
    pltpu.sync_copy(shared_v, all_v)

    def merge_body(j, carry):
        bvals, bidxs = carry
        take = lanes == j
        return (jnp.where(take, all_v[pl.ds(j * 2 * L, L)], bvals),
                jnp.where(take, all_v[pl.ds(j * 2 * L + L, L)], bidxs))

    vals16, idxf16 = lax.fori_loop(
        0, NS, merge_body,
        (jnp.full((L,), jnp.inf, jnp.float32), jnp.zeros((L,), jnp.float32)),
        unroll=4,
    )
    idxs16 = idxf16.astype(jnp.int32)
    gmin = jnp.min(vals16)
    gidx = jnp.min(jnp.where(vals16 == gmin, idxs16, I32_MAX))

    zeros_v = jnp.zeros((L,), jnp.float32)

    def fill_body(j, _):
        chunk_v[pl.ds(j * L, L)] = zeros_v
        return 0

    lax.fori_loop(0, SCAN // L, fill_body, 0, unroll=8)

    local = gidx - base
    owner = jnp.logical_and(local >= 0, local < SCAN)

    @pl.when(owner)
    def _():
        jhot = local // L
        lhot = local - jhot * L
        chunk_v[pl.ds(jhot * L, L)] = jnp.where(
            lanes == lhot, 1.0, 0.0).astype(jnp.float32)

    pltpu.sync_copy(chunk_v, out_hbm.at[pl.ds(base, SCAN)])


def kernel(x):
    return _stargmin_sc(x.reshape(N)).reshape(1, N)


# speedup vs baseline: 1.1181x; 1.0090x over previous
import functools

import jax
import jax.numpy as jnp
from jax import lax
from jax.experimental import pallas as pl
from jax.experimental.pallas import tpu as pltpu
from jax.experimental.pallas import tpu_sc as plsc

N = 32768
L = 16
NS = 16
SCAN = N // NS
HALF = SCAN // 2
I32_MAX = 2**31 - 1

_mesh = plsc.VectorSubcoreMesh(core_axis_name="c", subcore_axis_name="s",
                               num_cores=1)


@functools.partial(
    pl.kernel,
    out_type=jax.ShapeDtypeStruct((N,), jnp.float32),
    mesh=_mesh,
    compiler_params=pltpu.CompilerParams(needs_layout_passes=False),
    scratch_types=[
        pltpu.VMEM((SCAN,), jnp.float32),
        pltpu.VMEM((SCAN,), jnp.float32),
        pltpu.VMEM((2 * L,), jnp.float32),
        pltpu.VMEM((NS * 2 * L,), jnp.float32),
        pltpu.VMEM_SHARED((NS * 2 * L,), jnp.float32),
        pltpu.SemaphoreType.DMA,
        pltpu.SemaphoreType.DMA,
    ],
)
def _stargmin_sc(x_hbm, out_hbm, chunk_v, out_v, stage_v, all_v, shared_v,
                 sem_a, sem_b):
    sid = lax.axis_index("s")
    lanes = lax.iota(jnp.int32, L)
    base = sid * SCAN

    in_a = pltpu.async_copy(x_hbm.at[pl.ds(base, HALF)],
                            chunk_v.at[pl.ds(0, HALF)], sem_a)
    in_b = pltpu.async_copy(x_hbm.at[pl.ds(base + HALF, HALF)],
                            chunk_v.at[pl.ds(HALF, HALF)], sem_b)
    zeros_v = jnp.zeros((L,), jnp.float32)

    def zfill_body(j, _):
        out_v[pl.ds(j * L, L)] = zeros_v
        return 0

    lax.fori_loop(0, SCAN // L, zfill_body, 0, unroll=8)

    def scan_body(j, carry):
        vmin, vidx = carry
        v = chunk_v[pl.ds(j * L, L)]
        gidx = base + j * L + lanes
        take = v < vmin
        return jnp.where(take, v, vmin), jnp.where(take, gidx, vidx)

    init = (jnp.full((L,), jnp.inf, jnp.float32), jnp.zeros((L,), jnp.int32))
    in_a.wait()
    carry = lax.fori_loop(0, HALF // L, scan_body, init, unroll=8)
    in_b.wait()
    vmin, vidx = lax.fori_loop(HALF // L, SCAN // L, scan_body, carry,
                               unroll=8)

    mval = jnp.min(vmin)
    midx = jnp.min(jnp.where(vmin == mval, vidx, I32_MAX))

    stage_v[pl.ds(0, L)] = jnp.full((L,), mval, jnp.float32)
    stage_v[pl.ds(L, L)] = jnp.full((L,), midx, jnp.int32).astype(jnp.float32)
    pltpu.sync_copy(stage_v, shared_v.at[pl.ds(sid * 2 * L, 2 * L)])
    out_zero = pltpu.async_copy(out_v, out_hbm.at[pl.ds(base, SCAN)], sem_a)
    plsc.subcore_barrier()
    pltpu.sync_copy(shared_v, all_v)

    def merge_body(j, carry):
        bvals, bidxs = carry
        take = lanes == j
        return (jnp.where(take, all_v[pl.ds(j * 2 * L, L)], bvals),
                jnp.where(take, all_v[pl.ds(j * 2 * L + L, L)], bidxs))

    vals16, idxf16 = lax.fori_loop(
        0, NS, merge_body,
        (jnp.full((L,), jnp.inf, jnp.float32), jnp.zeros((L,), jnp.float32)),
        unroll=4,
    )
    idxs16 = idxf16.astype(jnp.int32)
    gmin = jnp.min(vals16)
    gidx = jnp.min(jnp.where(vals16 == gmin, idxs16, I32_MAX))

    out_zero.wait()
    local = gidx - base
    owner = jnp.logical_and(local >= 0, local < SCAN)

    @pl.when(owner)
    def _():
        jhot = local // L
        lhot = local - jhot * L
        stage_v[pl.ds(0, L)] = jnp.where(lanes == lhot, 1.0, 0.0).astype(
            jnp.float32)
        pltpu.sync_copy(stage_v.at[pl.ds(0, L)],
                        out_hbm.at[pl.ds(base + jhot * L, L)])


def kernel(x):
    return _stargmin_sc(x.reshape(N)).reshape(1, N)


# speedup vs baseline: 1.1261x; 1.0071x over previous
import functools

import jax
import jax.numpy as jnp
from jax import lax
from jax.experimental import pallas as pl
from jax.experimental.pallas import tpu as pltpu
from jax.experimental.pallas import tpu_sc as plsc

N = 32768
L = 16
NS = 16
SCAN = N // NS
HALF = SCAN // 2
I32_MAX = 2**31 - 1

_mesh = plsc.VectorSubcoreMesh(core_axis_name="c", subcore_axis_name="s",
                               num_cores=1)


@functools.partial(
    pl.kernel,
    out_type=jax.ShapeDtypeStruct((N,), jnp.float32),
    mesh=_mesh,
    compiler_params=pltpu.CompilerParams(needs_layout_passes=False),
    scratch_types=[
        pltpu.VMEM((SCAN,), jnp.float32),
        pltpu.VMEM((SCAN,), jnp.float32),
        pltpu.VMEM((2 * L,), jnp.float32),
        pltpu.VMEM((NS * 2 * L,), jnp.float32),
        pltpu.VMEM_SHARED((NS * 2 * L,), jnp.float32),
        pltpu.SemaphoreType.DMA,
        pltpu.SemaphoreType.DMA,
    ],
)
def _stargmin_sc(x_hbm, out_hbm, chunk_v, out_v, stage_v, all_v, shared_v,
                 sem_a, sem_b):
    sid = lax.axis_index("s")
    lanes = lax.iota(jnp.int32, L)
    base = sid * SCAN

    in_a = pltpu.async_copy(x_hbm.at[pl.ds(base, HALF)],
                            chunk_v.at[pl.ds(0, HALF)], sem_a)
    in_b = pltpu.async_copy(x_hbm.at[pl.ds(base + HALF, HALF)],
                            chunk_v.at[pl.ds(HALF, HALF)], sem_b)
    zeros_v = jnp.zeros((L,), jnp.float32)

    def zfill_body(j, _):
        out_v[pl.ds(j * L, L)] = zeros_v
        return 0

    lax.fori_loop(0, SCAN // L, zfill_body, 0, unroll=8)

    def scan_body(j, carry):
        vs, ts = carry
        new_vs, new_ts = [], []
        for c in range(4):
            t = j * 4 + c
            v = chunk_v[pl.ds(t * L, L)]
            take = v < vs[c]
            new_vs.append(jnp.where(take, v, vs[c]))
            new_ts.append(jnp.where(take, jnp.full((L,), 0, jnp.int32) + t,
                                    ts[c]))
        return tuple(new_vs), tuple(new_ts)

    init = (tuple(jnp.full((L,), jnp.inf, jnp.float32) for _ in range(4)),
            tuple(jnp.zeros((L,), jnp.int32) for _ in range(4)))
    in_a.wait()
    carry = lax.fori_loop(0, HALF // (4 * L), scan_body, init, unroll=4)
    in_b.wait()
    vs, ts = lax.fori_loop(HALF // (4 * L), SCAN // (4 * L), scan_body,
                           carry, unroll=4)

    def combine(av, at, bv, bt):
        better = jnp.logical_or(
            bv < av, jnp.logical_and(bv == av, bt < at))
        return jnp.where(better, bv, av), jnp.where(better, bt, at)

    v01, t01 = combine(vs[0], ts[0], vs[1], ts[1])
    v23, t23 = combine(vs[2], ts[2], vs[3], ts[3])
    vmin, tmin = combine(v01, t01, v23, t23)
    vidx = base + tmin * L + lanes

    mval = jnp.min(vmin)
    midx = jnp.min(jnp.where(vmin == mval, vidx, I32_MAX))

    stage_v[pl.ds(0, L)] = jnp.full((L,), mval, jnp.float32)
    stage_v[pl.ds(L, L)] = jnp.full((L,), midx, jnp.int32).astype(jnp.float32)
    pltpu.sync_copy(stage_v, shared_v.at[pl.ds(sid * 2 * L, 2 * L)])
    out_zero = pltpu.async_copy(out_v, out_hbm.at[pl.ds(base, SCAN)], sem_a)
    plsc.subcore_barrier()
    pltpu.sync_copy(shared_v, all_v)

    def merge_body(j, carry):
        bvals, bidxs = carry
        take = lanes == j
        return (jnp.where(take, all_v[pl.ds(j * 2 * L, L)], bvals),
                jnp.where(take, all_v[pl.ds(j * 2 * L + L, L)], bidxs))

    vals16, idxf16 = lax.fori_loop(
        0, NS, merge_body,
        (jnp.full((L,), jnp.inf, jnp.float32), jnp.zeros((L,), jnp.float32)),
        unroll=4,
    )
    idxs16 = idxf16.astype(jnp.int32)
    gmin = jnp.min(vals16)
    gidx = jnp.min(jnp.where(vals16 == gmin, idxs16, I32_MAX))

    out_zero.wait()
    local = gidx - base
    owner = jnp.logical_and(local >= 0, local < SCAN)

    @pl.when(owner)
    def _():
        jhot = local // L
        lhot = local - jhot * L
        stage_v[pl.ds(0, L)] = jnp.where(lanes == lhot, 1.0, 0.0).astype(
            jnp.float32)
        pltpu.sync_copy(stage_v.at[pl.ds(0, L)],
                        out_hbm.at[pl.ds(base + jhot * L, L)])


def kernel(x):
    return _stargmin_sc(x.reshape(N)).reshape(1, N)
